# Initial kernel scaffold; baseline (speedup 1.0000x reference)
import jax
import jax.numpy as jnp
from jax.experimental import pallas as pl


def kernel(t, pos_enc):
    raise NotImplementedError("write your pallas kernel here")


# speedup vs baseline: 4.0709x; 4.0709x over previous
import functools

import jax
import jax.numpy as jnp
from jax import lax
from jax.experimental import pallas as pl
from jax.experimental.pallas import tpu as pltpu
from jax.experimental.pallas import tpu_sc as plsc

MAXP = 100000
D = 64
LANES = 16
STREAM = 128
CHUNK = 512
NBUF = 2


def _sc_gather(t_flat, pos_enc, n_total):
    nw = 32
    b_per_w = n_total // nw
    n_chunks = b_per_w // CHUNK
    assert n_chunks % NBUF == 0 and n_chunks >= 2 * NBUF
    mesh = plsc.VectorSubcoreMesh(core_axis_name="c", subcore_axis_name="s")

    @functools.partial(
        pl.kernel,
        out_type=jax.ShapeDtypeStruct((n_total, D), jnp.float32),
        mesh=mesh,
        scratch_types=[
            pltpu.VMEM((NBUF, CHUNK), jnp.int32),
            pltpu.VMEM((NBUF, CHUNK, D), jnp.float32),
            pltpu.SemaphoreType.DMA,
            pltpu.SemaphoreType.DMA,
        ],
        compiler_params=pltpu.CompilerParams(use_tc_tiling_on_sc=False),
    )
    def k(t_hbm, table_hbm, out_hbm, idx_v, rows_v, gat_sem, out_sem):
        wid = lax.axis_index("s") * 2 + lax.axis_index("c")
        base = wid * b_per_w

        def load_idx(g, slot):
            pltpu.sync_copy(t_hbm.at[pl.ds(base + g * CHUNK, CHUNK)],
                            idx_v.at[slot])

            def adj(i, carry):
                v = idx_v[slot, pl.ds(i * LANES, LANES)]
                v = v - 1
                v = jnp.where(v < 0, MAXP - 1, v)
                idx_v[slot, pl.ds(i * LANES, LANES)] = v
                return carry

            lax.fori_loop(0, CHUNK // LANES, adj, 0, unroll=4)

        def fire_gather(slot):
            for j in range(CHUNK // STREAM):
                pltpu.async_copy(
                    table_hbm.at[idx_v.at[slot, pl.ds(j * STREAM, STREAM)]],
                    rows_v.at[slot, pl.ds(j * STREAM, STREAM)],
                    gat_sem)

        def drain_gather():
            for j in range(CHUNK // STREAM):
                pltpu.make_async_copy(
                    table_hbm.at[idx_v.at[0, pl.ds(0, STREAM)]],
                    rows_v.at[0, pl.ds(0, STREAM)],
                    gat_sem).wait()

        def fire_out(g, slot):
            pltpu.async_copy(rows_v.at[slot],
                             out_hbm.at[pl.ds(base + g * CHUNK, CHUNK)],
                             out_sem)

        def drain_out():
            pltpu.make_async_copy(rows_v.at[0],
                                  out_hbm.at[pl.ds(0, CHUNK)],
                                  out_sem).wait()

        def step(g, slot, nslot, do_load):
            if do_load:
                load_idx(g + 1, nslot)
            drain_gather()
            if do_load:
                fire_gather(nslot)
            fire_out(g, slot)
            drain_out()

        load_idx(0, 0)
        fire_gather(0)

        def outer(i, carry):
            for b in range(NBUF):
                step(i * NBUF + b, b, (b + 1) % NBUF, True)
            return carry

        lax.fori_loop(0, (n_chunks - NBUF) // NBUF, outer, 0)

        for b in range(NBUF):
            g = n_chunks - NBUF + b
            step(g, b, (b + 1) % NBUF, b + 1 < NBUF)

    return k(t_flat, pos_enc)


def kernel(t, pos_enc):
    n_total = t.shape[0] * t.shape[1]
    t_flat = t.reshape(n_total).astype(jnp.int32)
    out = _sc_gather(t_flat, pos_enc, n_total)
    return out.reshape(t.shape[0], t.shape[1], D)


# speedup vs baseline: 4.1062x; 1.0087x over previous
import functools

import jax
import jax.numpy as jnp
from jax import lax
from jax.experimental import pallas as pl
from jax.experimental.pallas import tpu as pltpu
from jax.experimental.pallas import tpu_sc as plsc

MAXP = 100000
D = 64
LANES = 16
STREAM = 128
CHUNK = 512
NBUF = 3


def _sc_gather(t_flat, pos_enc, n_total):
    nw = 32
    b_per_w = n_total // nw
    n_chunks = b_per_w // CHUNK
    assert n_chunks >= 2 * NBUF + 2
    mesh = plsc.VectorSubcoreMesh(core_axis_name="c", subcore_axis_name="s")

    @functools.partial(
        pl.kernel,
        out_type=jax.ShapeDtypeStruct((n_total, D), jnp.float32),
        mesh=mesh,
        scratch_types=[
            pltpu.VMEM((NBUF, CHUNK), jnp.int32),
            pltpu.VMEM((NBUF, CHUNK, D), jnp.float32),
            [pltpu.SemaphoreType.DMA] * NBUF,
            [pltpu.SemaphoreType.DMA] * NBUF,
        ],
        compiler_params=pltpu.CompilerParams(use_tc_tiling_on_sc=False),
    )
    def k(t_hbm, table_hbm, out_hbm, idx_v, rows_v, gat_sems, out_sems):
        wid = lax.axis_index("s") * 2 + lax.axis_index("c")
        base = wid * b_per_w

        def load_idx(g, slot):
            pltpu.sync_copy(t_hbm.at[pl.ds(base + g * CHUNK, CHUNK)],
                            idx_v.at[slot])

            def adj(i, carry):
                v = idx_v[slot, pl.ds(i * LANES, LANES)]
                v = v - 1
                v = jnp.where(v < 0, MAXP - 1, v)
                idx_v[slot, pl.ds(i * LANES, LANES)] = v
                return carry

            lax.fori_loop(0, CHUNK // LANES, adj, 0, unroll=4)

        def fire_gather(slot):
            for j in range(CHUNK // STREAM):
                pltpu.async_copy(
                    table_hbm.at[idx_v.at[slot, pl.ds(j * STREAM, STREAM)]],
                    rows_v.at[slot, pl.ds(j * STREAM, STREAM)],
                    gat_sems[slot])

        def drain_gather(slot):
            for j in range(CHUNK // STREAM):
                pltpu.make_async_copy(
                    table_hbm.at[idx_v.at[0, pl.ds(0, STREAM)]],
                    rows_v.at[0, pl.ds(0, STREAM)],
                    gat_sems[slot]).wait()

        def fire_out(g, slot):
            pltpu.async_copy(rows_v.at[slot],
                             out_hbm.at[pl.ds(base + g * CHUNK, CHUNK)],
                             out_sems[slot])

        def drain_out(slot):
            pltpu.make_async_copy(rows_v.at[0],
                                  out_hbm.at[pl.ds(0, CHUNK)],
                                  out_sems[slot]).wait()

        def step(g, slot, nslot, do_load, do_drain_out):
            if do_load:
                load_idx(g + 1, nslot)
                if do_drain_out:
                    drain_out(nslot)
                fire_gather(nslot)
            drain_gather(slot)
            fire_out(g, slot)

        load_idx(0, 0)
        fire_gather(0)

        for g in range(NBUF - 1):
            step(g, g % NBUF, (g + 1) % NBUF, True, False)

        groups = (n_chunks - (NBUF - 1) - (NBUF + 1)) // NBUF
        first = NBUF - 1

        def outer(i, carry):
            for b in range(NBUF):
                g = first + i * NBUF + b
                slot = (first + b) % NBUF
                step(g, slot, (slot + 1) % NBUF, True, True)
            return carry

        lax.fori_loop(0, groups, outer, 0)

        for g in range(first + groups * NBUF, n_chunks):
            step(g, g % NBUF, (g + 1) % NBUF, g + 1 < n_chunks, True)

        for g in range(n_chunks - NBUF, n_chunks):
            drain_out(g % NBUF)

    return k(t_flat, pos_enc)


def kernel(t, pos_enc):
    n_total = t.shape[0] * t.shape[1]
    t_flat = t.reshape(n_total).astype(jnp.int32)
    out = _sc_gather(t_flat, pos_enc, n_total)
    return out.reshape(t.shape[0], t.shape[1], D)
